# baseline (speedup 1.0000x reference)
import jax
import jax.numpy as jnp
from jax import lax
from jax.experimental import pallas as pl
from jax.experimental.pallas import tpu as pltpu
from jax.experimental.pallas import tpu_sc as plsc

_D = 32
_IDX_CHUNK = 128
_BLK = 2048


def _build_gather(B, mesh):
    NC, NS = mesh.num_cores, mesh.num_subcores
    NW = NC * NS
    bpw = B // NW
    nchunk = bpw // _IDX_CHUNK

    def body(uidx_hbm, iidx_hbm, utab_hbm, itab_hbm, ue_hbm, ie_hbm,
             uidx_v, iidx_v, urows_v, irows_v, sem):
        wid = lax.axis_index("s") * NC + lax.axis_index("c")
        base = wid * bpw
        pltpu.sync_copy(uidx_hbm.at[pl.ds(base, bpw)], uidx_v)
        pltpu.sync_copy(iidx_hbm.at[pl.ds(base, bpw)], iidx_v)
        copies = []
        for j in range(nchunk):
            sl = pl.ds(j * _IDX_CHUNK, _IDX_CHUNK)
            copies.append(
                pltpu.async_copy(utab_hbm.at[uidx_v.at[sl]], urows_v.at[sl], sem))
            copies.append(
                pltpu.async_copy(itab_hbm.at[iidx_v.at[sl]], irows_v.at[sl], sem))
        for c in copies:
            c.wait()
        pltpu.sync_copy(urows_v, ue_hbm.at[pl.ds(base, bpw)])
        pltpu.sync_copy(irows_v, ie_hbm.at[pl.ds(base, bpw)])

    out_t = (jax.ShapeDtypeStruct((B, _D), jnp.float32),
             jax.ShapeDtypeStruct((B, _D), jnp.float32))
    return pl.kernel(
        body, out_type=out_t, mesh=mesh,
        compiler_params=pltpu.CompilerParams(use_tc_tiling_on_sc=False),
        scratch_types=[
            pltpu.VMEM((bpw,), jnp.int32),
            pltpu.VMEM((bpw,), jnp.int32),
            pltpu.VMEM((bpw, _D), jnp.float32),
            pltpu.VMEM((bpw, _D), jnp.float32),
            pltpu.SemaphoreType.DMA,
        ])


def _mlp_body(ue_ref, ie_ref, w1a, w1b, b1r, w2r, b2r, w3r, b3r, out_ref):
    x = (jnp.dot(ue_ref[...], w1a[...], preferred_element_type=jnp.float32)
         + jnp.dot(ie_ref[...], w1b[...], preferred_element_type=jnp.float32)
         + b1r[...])
    h = jnp.maximum(x, 0.0)
    h = jnp.maximum(
        jnp.dot(h, w2r[...], preferred_element_type=jnp.float32) + b2r[...], 0.0)
    o = jnp.sum(h * w3r[...], axis=1) + b3r[0, 0]
    out_ref[...] = 1.0 / (1.0 + jnp.exp(-o))


def _mlp(ue, ie, W1a, W1b, b1, W2, b2, w3t, b3, B):
    grid = B // _BLK
    row_spec = pl.BlockSpec((_BLK, _D), lambda i: (i, 0))

    def full(shape):
        return pl.BlockSpec(shape, lambda i: tuple(0 for _ in shape))

    return pl.pallas_call(
        _mlp_body,
        grid=(grid,),
        in_specs=[
            row_spec, row_spec,
            full(W1a.shape), full(W1b.shape), full(b1.shape),
            full(W2.shape), full(b2.shape), full(w3t.shape), full(b3.shape),
        ],
        out_specs=pl.BlockSpec((_BLK,), lambda i: (i,)),
        out_shape=jax.ShapeDtypeStruct((B,), jnp.float32),
    )(ue, ie, W1a, W1b, b1, W2, b2, w3t, b3)


def kernel(user, item, user_table, item_table, W1, b1, W2, b2, W3, b3):
    B = user.shape[0]
    mesh = plsc.VectorSubcoreMesh(core_axis_name="c", subcore_axis_name="s")
    gather = _build_gather(B, mesh)
    ue, ie = gather(user.astype(jnp.int32), item.astype(jnp.int32),
                    user_table, item_table)
    W1a, W1b = W1[:_D], W1[_D:]
    return _mlp(ue, ie, W1a, W1b,
                b1.reshape(1, -1), W2, b2.reshape(1, -1),
                W3.reshape(1, -1), b3.reshape(1, 1), B)
